# baseline (speedup 1.0000x reference)
import functools

import jax
import jax.numpy as jnp
from jax import lax
from jax.experimental import pallas as pl
from jax.experimental.pallas import tpu as pltpu
from jax.experimental.pallas import tpu_sc as plsc

NC = 2
NS = 16
L = 16
NW = NC * NS

N_ROWS = 1_000_000
D = 64
N_UPD = 16384
MROWS = N_ROWS // 2
MD = 2 * D
ROWS_PER_W = N_ROWS // NW
MW = MROWS // NW
NCHUNK = N_UPD // L
STAGE = 128
GCH = 128
NGD = STAGE // GCH
SB = STAGE // L
OWN_CAP = N_UPD + L
CPR = 125
NCH = MW // CPR
PAIRS = (NCH + 1) // 2


def _body(x_hbm, idx_hbm, val_hbm, out_hbm,
          idx_all, pos_own, mpos_own, vals_stage, rb, acc_v, cp_a, cp_b,
          sem_in_a, sem_in_b, sem_out_a, sem_out_b, sem_g, sem_rmw):
  wid = lax.axis_index("s") * NC + lax.axis_index("c")
  lo = wid * ROWS_PER_W
  hi = lo + ROWS_PER_W
  lo_m = wid * MW
  iota = lax.iota(jnp.int32, L)

  pltpu.sync_copy(idx_hbm, idx_all)

  def _prefill(i, carry):
    pos_own[pl.ds(i * L, L)] = jnp.zeros((L,), jnp.int32)
    mpos_own[pl.ds(i * L, L)] = jnp.zeros((L,), jnp.int32)
    return carry
  lax.fori_loop(0, OWN_CAP // L, _prefill, 0)

  def _compact(i, off):
    v = idx_all[pl.ds(i * L, L)]
    m = (v >= lo) & (v < hi)
    pc = plsc.cumsum(m.astype(jnp.int32))
    dest = off + pc - 1
    pos = i * L + iota
    plsc.store_scatter(pos_own, [dest], pos, mask=m)
    plsc.store_scatter(mpos_own, [dest], lax.shift_right_logical(pos, 1),
                       mask=m)
    return off + jnp.max(pc)
  n_own = lax.fori_loop(0, NCHUNK, _compact, jnp.int32(0))

  n_batches = (n_own + L - 1) // L
  n_super = (n_batches + SB - 1) // SB

  def _fire_gathers(s):
    for g in range(NGD):
      pltpu.async_copy(
          val_hbm.at[mpos_own.at[pl.ds(s * STAGE + g * GCH, GCH)]],
          vals_stage.at[pl.ds(g * GCH, GCH)],
          sem_g,
      )

  def _drain_gathers(s):
    for g in range(NGD):
      pltpu.make_async_copy(
          val_hbm.at[mpos_own.at[pl.ds(s * STAGE + g * GCH, GCH)]],
          vals_stage.at[pl.ds(g * GCH, GCH)],
          sem_g,
      ).wait()

  _fire_gathers(0)

  def _ld(c, buf, sem):
    return pltpu.async_copy(x_hbm.at[pl.ds(lo_m + c * CPR, CPR)], buf, sem)

  def _ld_wait(c, buf, sem):
    pltpu.make_async_copy(
        x_hbm.at[pl.ds(lo_m + c * CPR, CPR)], buf, sem).wait()

  def _st(c, buf, sem):
    return pltpu.async_copy(
        buf, out_hbm.at[pl.ds(lo_m + c * CPR, CPR)], sem)

  def _st_wait(c, buf, sem):
    pltpu.make_async_copy(
        buf, out_hbm.at[pl.ds(lo_m + c * CPR, CPR)], sem).wait()

  _ld(0, cp_a, sem_in_a)

  def _copy_pair(j, carry):
    c0 = 2 * j
    c1 = c0 + 1

    @pl.when(j > 0)
    def _():
      _st_wait(c1 - 2, cp_b, sem_out_b)

    @pl.when(c1 < NCH)
    def _():
      _ld(c1, cp_b, sem_in_b)

    _ld_wait(c0, cp_a, sem_in_a)
    _st(c0, cp_a, sem_out_a)
    _st_wait(c0, cp_a, sem_out_a)

    @pl.when(c0 + 2 < NCH)
    def _():
      _ld(c0 + 2, cp_a, sem_in_a)

    @pl.when(c1 < NCH)
    def _():
      _ld_wait(c1, cp_b, sem_in_b)
      _st(c1, cp_b, sem_out_b)

    return carry
  lax.fori_loop(0, PAIRS, _copy_pair, 0)
  if NCH % 2 == 0:
    _st_wait(NCH - 1, cp_b, sem_out_b)

  def _super(s, carry):
    @pl.when(s > 0)
    def _():
      _fire_gathers(s)
    _drain_gathers(s)

    nb = jnp.minimum(SB, n_batches - s * SB)

    def _batch(b, carry2):
      base = s * STAGE + b * L
      valid = (base + iota) < n_own
      pos_vec = pos_own[pl.ds(base, L)]
      idxv = jnp.where(valid, plsc.load_gather(idx_all, [pos_vec]), lo)
      mv = lax.shift_right_logical(idxv, 1)
      half = lax.bitwise_and(idxv, 1)
      hp = lax.bitwise_and(pos_vec, 1)

      pltpu.async_copy(out_hbm.at[mv], rb, sem_rmw).wait()

      fs = iota
      for s_rot in range(1, L):
        perm = lax.rem(iota + (L - s_rot), L)
        shifted = jnp.take_along_axis(mv, perm, axis=0)
        eq = (mv == shifted) & (iota >= s_rot)
        fs = jnp.where(eq, jnp.minimum(fs, iota - s_rot), fs)

      def _col(c, carry3):
        cvec = jnp.full((L,), c, jnp.int32)
        plsc.store_scatter(acc_v, [iota, cvec], jnp.zeros((L,), jnp.float32))
        vc = cvec - D * half
        in_half = (vc >= 0) & (vc < D)
        sc = jnp.clip(vc + D * hp, 0, MD - 1)
        v_col = plsc.load_gather(vals_stage, [b * L + iota, sc])
        v_col = jnp.where(valid & in_half, v_col, jnp.float32(0))
        plsc.addupdate_scatter(acc_v, [fs, cvec], v_col)
        g_col = plsc.load_gather(rb, [iota, cvec])
        s_col = plsc.load_gather(acc_v, [fs, cvec])
        plsc.store_scatter(rb, [iota, cvec], g_col + s_col)
        return carry3
      lax.fori_loop(0, MD, _col, 0)

      pltpu.async_copy(rb, out_hbm.at[mv], sem_rmw).wait()
      return carry2

    lax.fori_loop(0, nb, _batch, 0)
    return carry

  lax.fori_loop(0, n_super, _super, 0)

  @pl.when(n_super == 0)
  def _():
    _drain_gathers(0)


_mesh = plsc.VectorSubcoreMesh(
    core_axis_name="c", subcore_axis_name="s", num_cores=NC, num_subcores=NS
)

_scatter_add = functools.partial(
    pl.kernel,
    out_type=jax.ShapeDtypeStruct((MROWS, MD), jnp.float32),
    mesh=_mesh,
    compiler_params=pltpu.CompilerParams(
        use_tc_tiling_on_sc=False, needs_layout_passes=False),
    scratch_types=[
        pltpu.VMEM((N_UPD,), jnp.int32),
        pltpu.VMEM((OWN_CAP,), jnp.int32),
        pltpu.VMEM((OWN_CAP,), jnp.int32),
        pltpu.VMEM((STAGE, MD), jnp.float32),
        pltpu.VMEM((L, MD), jnp.float32),
        pltpu.VMEM((L, MD), jnp.float32),
        pltpu.VMEM((CPR, MD), jnp.float32),
        pltpu.VMEM((CPR, MD), jnp.float32),
        pltpu.SemaphoreType.DMA,
        pltpu.SemaphoreType.DMA,
        pltpu.SemaphoreType.DMA,
        pltpu.SemaphoreType.DMA,
        pltpu.SemaphoreType.DMA,
        pltpu.SemaphoreType.DMA,
    ],
)(_body)


def kernel(x, indices, values, acc):
  del acc
  out = _scatter_add(
      x.reshape(MROWS, MD),
      indices.astype(jnp.int32),
      values.reshape(N_UPD // 2, MD),
  )
  return out.reshape(N_ROWS, D)


# speedup vs baseline: 1.2616x; 1.2616x over previous
import functools

import jax
import jax.numpy as jnp
from jax import lax
from jax.experimental import pallas as pl
from jax.experimental.pallas import tpu as pltpu
from jax.experimental.pallas import tpu_sc as plsc

NC = 2
NS = 16
L = 16
NW = NC * NS

N_ROWS = 1_000_000
D = 64
N_UPD = 16384
ROWS_PER_W = N_ROWS // NW
NCHUNK = N_UPD // L
OWN_CAP = N_UPD + L


def _body(idx_hbm, val_hbm, out_ref,
          idx_all, pos_own, vb, rb, acc_v, sem_g, sem_rmw):
  wid = lax.axis_index("s") * NC + lax.axis_index("c")
  lo = wid * ROWS_PER_W
  hi = lo + ROWS_PER_W
  iota = lax.iota(jnp.int32, L)

  pltpu.sync_copy(idx_hbm, idx_all)

  def _prefill(i, carry):
    pos_own[pl.ds(i * L, L)] = jnp.zeros((L,), jnp.int32)
    return carry
  lax.fori_loop(0, OWN_CAP // L, _prefill, 0)

  def _compact(i, off):
    v = idx_all[pl.ds(i * L, L)]
    m = (v >= lo) & (v < hi)
    pc = plsc.cumsum(m.astype(jnp.int32))
    dest = off + pc - 1
    plsc.store_scatter(pos_own, [dest], i * L + iota, mask=m)
    return off + jnp.max(pc)
  n_own = lax.fori_loop(0, NCHUNK, _compact, jnp.int32(0))

  n_batches = (n_own + L - 1) // L

  def _batch(b, carry):
    base = b * L
    valid = (base + iota) < n_own
    pos_vec = pos_own[pl.ds(base, L)]
    idxv = jnp.where(valid, plsc.load_gather(idx_all, [pos_vec]), lo)

    g_out = pltpu.async_copy(out_ref.at[idxv], rb, sem_rmw)
    pltpu.async_copy(val_hbm.at[pos_vec], vb, sem_g).wait()
    g_out.wait()

    fs = iota
    for s_rot in range(1, L):
      perm = lax.rem(iota + (L - s_rot), L)
      shifted = jnp.take_along_axis(idxv, perm, axis=0)
      eq = (idxv == shifted) & (iota >= s_rot)
      fs = jnp.where(eq, jnp.minimum(fs, iota - s_rot), fs)

    def _col(c, carry3):
      cvec = jnp.full((L,), c, jnp.int32)
      plsc.store_scatter(acc_v, [iota, cvec], jnp.zeros((L,), jnp.float32))
      v_col = plsc.load_gather(vb, [iota, cvec])
      v_col = jnp.where(valid, v_col, jnp.float32(0))
      plsc.addupdate_scatter(acc_v, [fs, cvec], v_col)
      g_col = plsc.load_gather(rb, [iota, cvec])
      s_col = plsc.load_gather(acc_v, [fs, cvec])
      plsc.store_scatter(rb, [iota, cvec], g_col + s_col)
      return carry3
    lax.fori_loop(0, D, _col, 0)

    pltpu.async_copy(rb, out_ref.at[idxv], sem_rmw).wait()
    return carry

  lax.fori_loop(0, n_batches, _batch, 0)


_mesh = plsc.VectorSubcoreMesh(
    core_axis_name="c", subcore_axis_name="s", num_cores=NC, num_subcores=NS
)

_rmw = functools.partial(
    pl.kernel,
    mesh=_mesh,
    compiler_params=pltpu.CompilerParams(
        use_tc_tiling_on_sc=False, needs_layout_passes=False),
    scratch_types=[
        pltpu.VMEM((N_UPD,), jnp.int32),
        pltpu.VMEM((OWN_CAP,), jnp.int32),
        pltpu.VMEM((L, D), jnp.float32),
        pltpu.VMEM((L, D), jnp.float32),
        pltpu.VMEM((L, D), jnp.float32),
        pltpu.SemaphoreType.DMA,
        pltpu.SemaphoreType.DMA,
    ],
)(_body)


def kernel(x, indices, values, acc):
  del acc
  out_ref = jax.new_ref(x)
  _rmw(indices.astype(jnp.int32), values, out_ref)
  return jax.freeze(out_ref)
